# Initial kernel scaffold; baseline (speedup 1.0000x reference)
import jax
import jax.numpy as jnp
from jax.experimental import pallas as pl


def kernel(disen_weight_att):
    raise NotImplementedError("write your pallas kernel here")


# speedup vs baseline: 2.2931x; 2.2931x over previous
import jax
import jax.numpy as jnp
from jax.experimental import pallas as pl
from jax.experimental.pallas import tpu as pltpu

_NF = 8
_C = 4096
_EPS = 1e-8
_T = 256
_B = _C // _T
_CORES = 2
_STEPS = (_B * _B) // _CORES


def _acc_kernel(x_ref, xt_ref, m_ref, g_ref):
    c = pl.program_id(0)
    s = pl.program_id(1)

    @pl.when(s == 0)
    def _init():
        m_ref[...] = jnp.zeros_like(m_ref)
        g_ref[...] = jnp.zeros_like(g_ref)

    idx = c * _STEPS + s
    pb = idx // _B
    qb = idx % _B
    po = pl.multiple_of(pb * _T, _T)
    qo = pl.multiple_of(qb * _T, _T)

    a = []
    for f in range(_NF):
        xrow = x_ref[f:f + 1, pl.ds(qo, _T)]
        xcol = xt_ref[pl.ds(po, _T), f:f + 1]
        diff = xcol - xrow
        a.append(jnp.sqrt(diff * diff + _EPS))

    cs = jnp.concatenate(
        [jnp.sum(a[f], axis=0, keepdims=True) for f in range(_NF)], axis=0)
    m_ref[:, pl.ds(qo, _T)] += cs

    for f in range(_NF):
        pcs = jnp.concatenate(
            [jnp.sum(a[f] * a[g], axis=0, keepdims=True)
             for g in range(f, _NF)], axis=1)
        g_ref[f:f + 1, f * _T:] += pcs


def _fin_kernel(m_ref, g8_ref, o_ref):
    m = m_ref[...]
    g8 = g8_ref[...]
    c2 = float(_C) * float(_C)
    ri = jax.lax.broadcasted_iota(jnp.int32, (_NF, _NF), 0)
    ci = jax.lax.broadcasted_iota(jnp.int32, (_NF, _NF), 1)
    msum = jnp.sum(m, axis=1, keepdims=True)
    m8 = jnp.concatenate(
        [jnp.sum(m * m[g:g + 1, :], axis=1, keepdims=True)
         for g in range(_NF)], axis=1)
    t = msum / c2
    trow = jnp.sum(jnp.where(ri == ci, t, 0.0), axis=0, keepdims=True)
    ttt = t * trow
    s8 = g8 * (1.0 / c2) - m8 * (2.0 / (c2 * float(_C))) + ttt
    dcov = jnp.sqrt(jnp.maximum(s8, 0.0) + _EPS)
    dmat = jnp.where(ri == ci, dcov, 0.0)
    dcol = jnp.sum(dmat, axis=1, keepdims=True)
    drow = jnp.sum(dmat, axis=0, keepdims=True)
    ratio = dcov / jnp.sqrt(dcol * drow + _EPS)
    o_ref[...] = jnp.sum(jnp.where(ci > ri, ratio, 0.0),
                         axis=(0, 1), keepdims=True)


def kernel(disen_weight_att):
    x = disen_weight_att.astype(jnp.float32)
    xt = x.T

    m_parts, g_parts = pl.pallas_call(
        _acc_kernel,
        grid=(_CORES, _STEPS),
        in_specs=[
            pl.BlockSpec((_NF, _C), lambda c, s: (0, 0)),
            pl.BlockSpec((_C, _NF), lambda c, s: (0, 0)),
        ],
        out_specs=[
            pl.BlockSpec((_NF, _C), lambda c, s: (c, 0)),
            pl.BlockSpec((_NF, _NF * _T), lambda c, s: (c, 0)),
        ],
        out_shape=[
            jax.ShapeDtypeStruct((_CORES * _NF, _C), jnp.float32),
            jax.ShapeDtypeStruct((_CORES * _NF, _NF * _T), jnp.float32),
        ],
        compiler_params=pltpu.CompilerParams(
            dimension_semantics=("parallel", "arbitrary")),
    )(x, xt)

    m = m_parts[:_NF] + m_parts[_NF:]
    g8 = g_parts.reshape(_CORES, _NF, _NF, _T).sum(axis=(0, 3))

    out = pl.pallas_call(
        _fin_kernel,
        out_shape=jax.ShapeDtypeStruct((1, 1), jnp.float32),
    )(m, g8)
    return out.reshape(())


# speedup vs baseline: 2.6453x; 1.1536x over previous
import jax
import jax.numpy as jnp
from jax.experimental import pallas as pl
from jax.experimental.pallas import tpu as pltpu

_NF = 8
_C = 4096
_EPS = 1e-8
_T = 512
_B = _C // _T
_CORES = 2
_STEPS = (_B * _B) // _CORES


def _acc_kernel(x_ref, xt_ref, m_ref, g_ref):
    c = pl.program_id(0)
    s = pl.program_id(1)

    @pl.when(s == 0)
    def _init():
        m_ref[...] = jnp.zeros_like(m_ref)
        g_ref[...] = jnp.zeros_like(g_ref)

    idx = s * _CORES + c
    pb = idx // _B
    qb = idx % _B
    po = pl.multiple_of(pb * _T, _T)
    qo = pl.multiple_of(qb * _T, _T)

    a = []
    for f in range(_NF):
        xrow = x_ref[f:f + 1, pl.ds(qo, _T)]
        xcol = xt_ref[pl.ds(po, _T), f:f + 1]
        diff = xcol - xrow
        a.append(jnp.sqrt(diff * diff + _EPS))

    cs = jnp.concatenate(
        [jnp.sum(a[f], axis=0, keepdims=True) for f in range(_NF)], axis=0)
    m_ref[:, pl.ds(qo, _T)] += cs

    @pl.when(pb <= qb)
    def _products():
        w = jnp.where(pb == qb, 1.0, 2.0).astype(jnp.float32)
        for f in range(_NF):
            pcs = jnp.concatenate(
                [jnp.sum(a[f] * a[g], axis=0, keepdims=True)
                 for g in range(f, _NF)], axis=1)
            g_ref[f:f + 1, f * _T:] += w * pcs


def _fin_kernel(m_ref, g8_ref, o_ref):
    m = m_ref[...]
    g8 = g8_ref[...]
    c2 = float(_C) * float(_C)
    ri = jax.lax.broadcasted_iota(jnp.int32, (_NF, _NF), 0)
    ci = jax.lax.broadcasted_iota(jnp.int32, (_NF, _NF), 1)
    msum = jnp.sum(m, axis=1, keepdims=True)
    m8 = jnp.concatenate(
        [jnp.sum(m * m[g:g + 1, :], axis=1, keepdims=True)
         for g in range(_NF)], axis=1)
    t = msum / c2
    trow = jnp.sum(jnp.where(ri == ci, t, 0.0), axis=0, keepdims=True)
    ttt = t * trow
    s8 = g8 * (1.0 / c2) - m8 * (2.0 / (c2 * float(_C))) + ttt
    dcov = jnp.sqrt(jnp.maximum(s8, 0.0) + _EPS)
    dmat = jnp.where(ri == ci, dcov, 0.0)
    dcol = jnp.sum(dmat, axis=1, keepdims=True)
    drow = jnp.sum(dmat, axis=0, keepdims=True)
    ratio = dcov / jnp.sqrt(dcol * drow + _EPS)
    o_ref[...] = jnp.sum(jnp.where(ci > ri, ratio, 0.0),
                         axis=(0, 1), keepdims=True)


def kernel(disen_weight_att):
    x = disen_weight_att.astype(jnp.float32)
    xt = x.T

    m_parts, g_parts = pl.pallas_call(
        _acc_kernel,
        grid=(_CORES, _STEPS),
        in_specs=[
            pl.BlockSpec((_NF, _C), lambda c, s: (0, 0)),
            pl.BlockSpec((_C, _NF), lambda c, s: (0, 0)),
        ],
        out_specs=[
            pl.BlockSpec((_NF, _C), lambda c, s: (c, 0)),
            pl.BlockSpec((_NF, _NF * _T), lambda c, s: (c, 0)),
        ],
        out_shape=[
            jax.ShapeDtypeStruct((_CORES * _NF, _C), jnp.float32),
            jax.ShapeDtypeStruct((_CORES * _NF, _NF * _T), jnp.float32),
        ],
        compiler_params=pltpu.CompilerParams(
            dimension_semantics=("arbitrary", "arbitrary")),
    )(x, xt)

    m = m_parts[:_NF] + m_parts[_NF:]
    g8 = g_parts.reshape(_CORES, _NF, _NF, _T).sum(axis=(0, 3))

    out = pl.pallas_call(
        _fin_kernel,
        out_shape=jax.ShapeDtypeStruct((1, 1), jnp.float32),
    )(m, g8)
    return out.reshape(())


# speedup vs baseline: 4.4228x; 1.6720x over previous
import functools

import jax
import jax.numpy as jnp
import numpy as np
from jax.experimental import pallas as pl
from jax.experimental.pallas import tpu as pltpu
from jax.sharding import Mesh, PartitionSpec as P

_NF = 8
_C = 4096
_EPS = 1e-8
_T = 256
_B = _C // _T
_TILES = _B * _B


def _acc_kernel(x_ref, xt_ref, ci_ref, m_ref, g_ref, *, nshard):
    s = pl.program_id(0)

    @pl.when(s == 0)
    def _init():
        m_ref[...] = jnp.zeros_like(m_ref)
        g_ref[...] = jnp.zeros_like(g_ref)

    idx = s * nshard + ci_ref[0, 0]
    pb = idx // _B
    qb = idx % _B
    po = pl.multiple_of(pb * _T, _T)
    qo = pl.multiple_of(qb * _T, _T)

    a = []
    for f in range(_NF):
        xrow = x_ref[f:f + 1, pl.ds(qo, _T)]
        xcol = xt_ref[pl.ds(po, _T), f:f + 1]
        diff = xcol - xrow
        a.append(jnp.sqrt(diff * diff + _EPS))

    cs = jnp.concatenate(
        [jnp.sum(a[f], axis=0, keepdims=True) for f in range(_NF)], axis=0)
    m_ref[:, pl.ds(qo, _T)] += cs

    @pl.when(pb <= qb)
    def _products():
        w = jnp.where(pb == qb, 1.0, 2.0).astype(jnp.float32)
        for f in range(_NF):
            pcs = jnp.concatenate(
                [jnp.sum(a[f] * a[g], axis=0, keepdims=True)
                 for g in range(f, _NF)], axis=1)
            g_ref[f:f + 1, f * _T:] += w * pcs


def _fin_kernel(m_ref, g8_ref, o_ref):
    m = m_ref[...]
    g8 = g8_ref[...]
    c2 = float(_C) * float(_C)
    ri = jax.lax.broadcasted_iota(jnp.int32, (_NF, _NF), 0)
    ci = jax.lax.broadcasted_iota(jnp.int32, (_NF, _NF), 1)
    msum = jnp.sum(m, axis=1, keepdims=True)
    m8 = jnp.concatenate(
        [jnp.sum(m * m[g:g + 1, :], axis=1, keepdims=True)
         for g in range(_NF)], axis=1)
    t = msum / c2
    trow = jnp.sum(jnp.where(ri == ci, t, 0.0), axis=0, keepdims=True)
    ttt = t * trow
    s8 = g8 * (1.0 / c2) - m8 * (2.0 / (c2 * float(_C))) + ttt
    dcov = jnp.sqrt(jnp.maximum(s8, 0.0) + _EPS)
    dmat = jnp.where(ri == ci, dcov, 0.0)
    dcol = jnp.sum(dmat, axis=1, keepdims=True)
    drow = jnp.sum(dmat, axis=0, keepdims=True)
    ratio = dcov / jnp.sqrt(dcol * drow + _EPS)
    o_ref[...] = jnp.sum(jnp.where(ci > ri, ratio, 0.0),
                         axis=(0, 1), keepdims=True)


def _acc_call(xs, xts, civ, *, nshard):
    return pl.pallas_call(
        functools.partial(_acc_kernel, nshard=nshard),
        grid=(_TILES // nshard,),
        in_specs=[
            pl.BlockSpec((_NF, _C), lambda s: (0, 0)),
            pl.BlockSpec((_C, _NF), lambda s: (0, 0)),
            pl.BlockSpec(memory_space=pltpu.SMEM),
        ],
        out_specs=[
            pl.BlockSpec((_NF, _C), lambda s: (0, 0)),
            pl.BlockSpec((_NF, _NF * _T), lambda s: (0, 0)),
        ],
        out_shape=[
            jax.ShapeDtypeStruct((_NF, _C), jnp.float32),
            jax.ShapeDtypeStruct((_NF, _NF * _T), jnp.float32),
        ],
        compiler_params=pltpu.CompilerParams(
            dimension_semantics=("arbitrary",)),
    )(xs, xts, civ)


def kernel(disen_weight_att):
    x = disen_weight_att.astype(jnp.float32)
    xt = x.T
    nshard = 2 if jax.device_count() >= 2 else 1
    mesh = Mesh(np.array(jax.devices()[:nshard]), ("c",))

    def _shard(xs, xts):
        ci = jax.lax.axis_index("c").astype(jnp.int32)
        civ = jnp.full((1, 1), ci, jnp.int32)
        m_p, g_p = _acc_call(xs, xts, civ, nshard=nshard)
        m = jax.lax.psum(m_p, "c")
        g = jax.lax.psum(g_p, "c")
        g8 = g.reshape(_NF, _NF, _T).sum(axis=-1)
        return pl.pallas_call(
            _fin_kernel,
            out_shape=jax.ShapeDtypeStruct((1, 1), jnp.float32),
        )(m, g8)

    out = jax.shard_map(
        _shard, mesh=mesh,
        in_specs=(P(None, None), P(None, None)),
        out_specs=P(None, None),
        check_vma=False,
    )(x, xt)
    return out.reshape(())


# speedup vs baseline: 5.7477x; 1.2996x over previous
import functools

import jax
import jax.numpy as jnp
import numpy as np
from jax.experimental import pallas as pl
from jax.experimental.pallas import tpu as pltpu
from jax.sharding import Mesh, PartitionSpec as P

_NF = 8
_C = 4096
_EPS = 1e-8
_T = 256
_B = _C // _T
_TILES = _B * _B


def _acc_kernel(x_ref, xt_ref, ci_ref, m_ref, g_ref, *, nshard):
    s = pl.program_id(0)

    @pl.when(s == 0)
    def _init():
        m_ref[...] = jnp.zeros_like(m_ref)
        g_ref[...] = jnp.zeros_like(g_ref)

    idx = s * nshard + ci_ref[0, 0]
    pb = idx // _B
    qb = idx % _B
    po = pl.multiple_of(pb * _T, _T)
    qo = pl.multiple_of(qb * _T, _T)

    a = []
    for f in range(_NF):
        xrow = x_ref[f:f + 1, pl.ds(qo, _T)]
        xcol = xt_ref[pl.ds(po, _T), f:f + 1]
        a.append(jnp.abs(xcol - xrow))

    cs = jnp.concatenate(
        [jnp.sum(a[f], axis=0, keepdims=True) for f in range(_NF)], axis=0)
    m_ref[:, pl.ds(qo, _T)] += cs

    @pl.when(pb <= qb)
    def _products():
        w = jnp.where(pb == qb, 1.0, 2.0).astype(jnp.float32)
        for f in range(_NF):
            pcs = jnp.concatenate(
                [jnp.sum(a[f] * a[g], axis=0, keepdims=True)
                 for g in range(f, _NF)], axis=1)
            g_ref[f:f + 1, f * _T:] += w * pcs


def _fin_kernel(m_ref, g8_ref, o_ref):
    m = m_ref[...]
    g8 = g8_ref[...]
    c2 = float(_C) * float(_C)
    ri = jax.lax.broadcasted_iota(jnp.int32, (_NF, _NF), 0)
    ci = jax.lax.broadcasted_iota(jnp.int32, (_NF, _NF), 1)
    msum = jnp.sum(m, axis=1, keepdims=True)
    m8 = jnp.concatenate(
        [jnp.sum(m * m[g:g + 1, :], axis=1, keepdims=True)
         for g in range(_NF)], axis=1)
    t = msum / c2
    trow = jnp.sum(jnp.where(ri == ci, t, 0.0), axis=0, keepdims=True)
    ttt = t * trow
    s8 = g8 * (1.0 / c2) - m8 * (2.0 / (c2 * float(_C))) + ttt
    dcov = jnp.sqrt(jnp.maximum(s8, 0.0) + _EPS)
    dmat = jnp.where(ri == ci, dcov, 0.0)
    dcol = jnp.sum(dmat, axis=1, keepdims=True)
    drow = jnp.sum(dmat, axis=0, keepdims=True)
    ratio = dcov / jnp.sqrt(dcol * drow + _EPS)
    o_ref[...] = jnp.sum(jnp.where(ci > ri, ratio, 0.0),
                         axis=(0, 1), keepdims=True)


def _acc_call(xs, xts, civ, *, nshard):
    return pl.pallas_call(
        functools.partial(_acc_kernel, nshard=nshard),
        grid=(_TILES // nshard,),
        in_specs=[
            pl.BlockSpec((_NF, _C), lambda s: (0, 0)),
            pl.BlockSpec((_C, _NF), lambda s: (0, 0)),
            pl.BlockSpec(memory_space=pltpu.SMEM),
        ],
        out_specs=[
            pl.BlockSpec((_NF, _C), lambda s: (0, 0)),
            pl.BlockSpec((_NF, _NF * _T), lambda s: (0, 0)),
        ],
        out_shape=[
            jax.ShapeDtypeStruct((_NF, _C), jnp.float32),
            jax.ShapeDtypeStruct((_NF, _NF * _T), jnp.float32),
        ],
        compiler_params=pltpu.CompilerParams(
            dimension_semantics=("arbitrary",)),
    )(xs, xts, civ)


def kernel(disen_weight_att):
    x = disen_weight_att.astype(jnp.float32)
    xt = x.T
    nshard = 2 if jax.device_count() >= 2 else 1
    mesh = Mesh(np.array(jax.devices()[:nshard]), ("c",))

    def _shard(xs, xts):
        ci = jax.lax.axis_index("c").astype(jnp.int32)
        civ = jnp.full((1, 1), ci, jnp.int32)
        m_p, g_p = _acc_call(xs, xts, civ, nshard=nshard)
        m = jax.lax.psum(m_p, "c")
        g = jax.lax.psum(g_p, "c")
        g8 = g.reshape(_NF, _NF, _T).sum(axis=-1)
        return pl.pallas_call(
            _fin_kernel,
            out_shape=jax.ShapeDtypeStruct((1, 1), jnp.float32),
        )(m, g8)

    out = jax.shard_map(
        _shard, mesh=mesh,
        in_specs=(P(None, None), P(None, None)),
        out_specs=P(None, None),
        check_vma=False,
    )(x, xt)
    return out.reshape(())
